# Initial kernel scaffold; baseline (speedup 1.0000x reference)
import jax
import jax.numpy as jnp
from jax.experimental import pallas as pl


def kernel(idx_20, idx_50, idx_200, table_20, table_50, table_200):
    raise NotImplementedError("write your pallas kernel here")


# speedup vs baseline: 3.8902x; 3.8902x over previous
import functools

import jax
import jax.numpy as jnp
from jax import lax
from jax.experimental import pallas as pl
from jax.experimental.pallas import tpu as pltpu
from jax.experimental.pallas import tpu_sc as plsc

EMBEDDING_DIM = 128
NUM_CORES = 2
NUM_SUBCORES = 16
NUM_WORKERS = NUM_CORES * NUM_SUBCORES
CHUNK = 128


def _gather_kernel(idx20, idx50, idx200, t20, t50, t200):
    n20, n50, n200 = idx20.shape[0], idx50.shape[0], idx200.shape[0]
    mesh = plsc.VectorSubcoreMesh(core_axis_name="c", subcore_axis_name="s")
    out_type = (
        jax.ShapeDtypeStruct((n20, EMBEDDING_DIM), jnp.float32),
        jax.ShapeDtypeStruct((n50, EMBEDDING_DIM), jnp.float32),
        jax.ShapeDtypeStruct((n200, EMBEDDING_DIM), jnp.float32),
    )

    @functools.partial(
        pl.kernel,
        out_type=out_type,
        mesh=mesh,
        scratch_types=[
            pltpu.VMEM((CHUNK,), jnp.int32),
            pltpu.VMEM((CHUNK, EMBEDDING_DIM), jnp.float32),
            pltpu.SemaphoreType.DMA,
        ],
    )
    def body(i20, i50, i200, tb20, tb50, tb200, o20, o50, o200,
             idx_v, rows_v, sem):
        wid = lax.axis_index("s") * NUM_CORES + lax.axis_index("c")
        for idx_hbm, t_hbm, out_hbm, n in (
            (i20, tb20, o20, n20),
            (i50, tb50, o50, n50),
            (i200, tb200, o200, n200),
        ):
            per_w = n // NUM_WORKERS
            base = wid * per_w
            nch = per_w // CHUNK

            def chunk_body(i, _, idx_hbm=idx_hbm, t_hbm=t_hbm,
                           out_hbm=out_hbm, base=base):
                off = base + i * CHUNK
                pltpu.sync_copy(idx_hbm.at[pl.ds(off, CHUNK)], idx_v)
                pltpu.async_copy(t_hbm.at[idx_v], rows_v, sem).wait()
                pltpu.sync_copy(rows_v, out_hbm.at[pl.ds(off, CHUNK)])
                return 0

            lax.fori_loop(0, nch, chunk_body, 0)

    return body(idx20, idx50, idx200, t20, t50, t200)


def kernel(idx_20, idx_50, idx_200, table_20, table_50, table_200):
    b20, l20 = idx_20.shape
    b50, l50 = idx_50.shape
    b200, l200 = idx_200.shape
    o20, o50, o200 = _gather_kernel(
        idx_20.reshape(-1).astype(jnp.int32),
        idx_50.reshape(-1).astype(jnp.int32),
        idx_200.reshape(-1).astype(jnp.int32),
        table_20, table_50, table_200,
    )
    return (
        o20.reshape(b20, l20, EMBEDDING_DIM),
        o50.reshape(b50, l50, EMBEDDING_DIM),
        o200.reshape(b200, l200, EMBEDDING_DIM),
    )


# speedup vs baseline: 5.7934x; 1.4892x over previous
import functools

import jax
import jax.numpy as jnp
from jax import lax
from jax.experimental import pallas as pl
from jax.experimental.pallas import tpu as pltpu
from jax.experimental.pallas import tpu_sc as plsc

D = 128
NUM_CORES = 2
NUM_SUBCORES = 16
NW = NUM_CORES * NUM_SUBCORES
C = 128
PATHS = (20, 50, 200)
TOTAL_CHUNKS = sum(PATHS)


def _gather_kernel(idx_all, t20, t50, t200, n20, n50, n200):
    mesh = plsc.VectorSubcoreMesh(core_axis_name="c", subcore_axis_name="s")
    out_type = (
        jax.ShapeDtypeStruct((n20, D), jnp.float32),
        jax.ShapeDtypeStruct((n50, D), jnp.float32),
        jax.ShapeDtypeStruct((n200, D), jnp.float32),
    )

    @functools.partial(
        pl.kernel,
        out_type=out_type,
        mesh=mesh,
        scratch_types=[
            pltpu.VMEM((TOTAL_CHUNKS, C), jnp.int32),
            pltpu.VMEM((C, D), jnp.float32),
            pltpu.VMEM((C, D), jnp.float32),
            pltpu.SemaphoreType.DMA,
            pltpu.SemaphoreType.DMA,
            pltpu.SemaphoreType.DMA,
            pltpu.SemaphoreType.DMA,
        ],
    )
    def body(idx_hbm, tb20, tb50, tb200, o20, o50, o200,
             idx_v, rows0, rows1, g0, g1, w0, w1):
        wid = lax.axis_index("s") * NUM_CORES + lax.axis_index("c")
        rows = (rows0, rows1)
        gsem = (g0, g1)
        wsem = (w0, w1)

        pltpu.sync_copy(idx_hbm.at[wid], idx_v)

        tcb = 0
        for t_hbm, out_hbm, ct in ((tb20, o20, PATHS[0]),
                                   (tb50, o50, PATHS[1]),
                                   (tb200, o200, PATHS[2])):
            base_out = wid * ct * C

            def fire_gather(j, b, t_hbm=t_hbm, tcb=tcb):
                pltpu.async_copy(t_hbm.at[idx_v.at[tcb + j]], rows[b],
                                 gsem[b])

            def wait_gather(b, t_hbm=t_hbm):
                pltpu.make_async_copy(t_hbm.at[idx_v.at[tcb]], rows[b],
                                      gsem[b]).wait()

            def fire_write(j, b, out_hbm=out_hbm, base_out=base_out):
                pltpu.async_copy(rows[b],
                                 out_hbm.at[pl.ds(base_out + j * C, C)],
                                 wsem[b])

            def wait_write(b, out_hbm=out_hbm, base_out=base_out):
                pltpu.make_async_copy(rows[b],
                                      out_hbm.at[pl.ds(base_out, C)],
                                      wsem[b]).wait()

            fire_gather(0, 0)
            fire_gather(1, 1)

            def pair(gi, _):
                for b in (0, 1):
                    j = gi * 2 + b
                    wait_gather(b)
                    fire_write(j, b)
                    wait_write(b)
                    fire_gather(j + 2, b)
                return 0

            lax.fori_loop(0, (ct - 2) // 2, pair, 0)

            for b in (0, 1):
                j = ct - 2 + b
                wait_gather(b)
                fire_write(j, b)
                wait_write(b)
            tcb += ct

    return body(idx_all, t20, t50, t200)


def kernel(idx_20, idx_50, idx_200, table_20, table_50, table_200):
    b20, l20 = idx_20.shape
    b50, l50 = idx_50.shape
    b200, l200 = idx_200.shape
    idx_all = jnp.concatenate(
        [
            idx_20.reshape(NW, PATHS[0], C).astype(jnp.int32),
            idx_50.reshape(NW, PATHS[1], C).astype(jnp.int32),
            idx_200.reshape(NW, PATHS[2], C).astype(jnp.int32),
        ],
        axis=1,
    )
    o20, o50, o200 = _gather_kernel(
        idx_all, table_20, table_50, table_200,
        b20 * l20, b50 * l50, b200 * l200,
    )
    return (
        o20.reshape(b20, l20, D),
        o50.reshape(b50, l50, D),
        o200.reshape(b200, l200, D),
    )
